# baseline (speedup 1.0000x reference)
import functools

import jax
import jax.numpy as jnp
from jax import lax
from jax.experimental import pallas as pl
from jax.experimental.pallas import tpu as pltpu
from jax.experimental.pallas import tpu_sc as plsc

N = 10000
NP = 10240
SLICE = NP // 16
H1 = 16
L = 16
ZR = 40

_MESH = dict(core_axis_name="c", subcore_axis_name="s")
_SC_PARAMS = pltpu.CompilerParams(use_tc_tiling_on_sc=False)
_SC_PARAMS_NOLAYOUT = pltpu.CompilerParams(use_tc_tiling_on_sc=False,
                                           needs_layout_passes=False)


def _fill(ref, n, value):
    def body(i, _):
        ref[pl.ds(i * L, L)] = jnp.full((L,), value, jnp.float32)
        return 0
    lax.fori_loop(0, n // L, body, 0)


def _tc1_body(x_ref, w_ref, h_ref):
    h_ref[...] = jnp.dot(x_ref[...], w_ref[...],
                         preferred_element_type=jnp.float32)


def _tc1(xp, w1):
    blk = 1024
    return pl.pallas_call(
        _tc1_body,
        grid=(NP // blk,),
        in_specs=[
            pl.BlockSpec((blk, xp.shape[1]), lambda i: (i, 0)),
            pl.BlockSpec((xp.shape[1], H1), lambda i: (0, 0)),
        ],
        out_specs=pl.BlockSpec((blk, H1), lambda i: (i, 0)),
        out_shape=jax.ShapeDtypeStruct((NP, H1), jnp.float32),
    )(xp, w1)


def _rsqrt16(x):
    i = plsc.bitcast(x, jnp.int32)
    y = plsc.bitcast(jnp.int32(0x5F3759DF) - (i >> 1), jnp.float32)
    for _ in range(3):
        y = y * (1.5 - 0.5 * x * y * y)
    return y


def _agg1_body(e32, ew, wv, h_hbm, src_hbm, dst_hbm, p0_hbm, p1_hbm, dis_hbm,
               g1_sp, acc_sp, deg_sp, zbuf, zvec, ones, idxb, hbuf, dvec,
               sidx, didx, rows, gsem):
    cid = lax.axis_index("c")
    sid = lax.axis_index("s")
    sl = pl.ds(sid * SLICE, SLICE)

    def z(i, _):
        zbuf[i] = jnp.zeros((L,), jnp.float32)
        return 0
    lax.fori_loop(0, ZR, z, 0)
    _fill(zvec, SLICE, 0.0)
    _fill(ones, ew, 1.0)
    for k in range(SLICE // ZR):
        pltpu.sync_copy(zbuf, acc_sp.at[pl.ds(sid * SLICE + k * ZR, ZR)])
    pltpu.sync_copy(zvec, deg_sp.at[sl])

    pltpu.sync_copy(dst_hbm.at[pl.ds(sid * ew, ew)], idxb)
    plsc.subcore_barrier()
    pltpu.sync_copy(ones, deg_sp.at[idxb], add=True)
    plsc.subcore_barrier()

    pltpu.sync_copy(deg_sp.at[sl], dvec)
    pltpu.sync_copy(h_hbm.at[sl], hbuf)

    def mkdis(i, _):
        s = pl.ds(i * L, L)
        dvec[s] = _rsqrt16(dvec[s] + 1.0)
        return 0
    lax.fori_loop(0, SLICE // L, mkdis, 0)

    @pl.when(cid == 0)
    def _():
        pltpu.sync_copy(dvec, dis_hbm.at[sl])

    def scale(r, _):
        d = plsc.load_gather(dvec, [jnp.full((L,), r, jnp.int32)])
        hbuf[r] = hbuf[r] * d
        return 0
    lax.fori_loop(0, SLICE, scale, 0)
    pltpu.sync_copy(hbuf, g1_sp.at[sl])
    plsc.subcore_barrier()

    base = (cid * 16 + sid) * e32
    for j in range(e32 // wv):
        pltpu.sync_copy(src_hbm.at[pl.ds(base + j * wv, wv)], sidx.at[j])
        pltpu.sync_copy(dst_hbm.at[pl.ds(base + j * wv, wv)], didx.at[j])
    for j in range(e32 // wv):
        pltpu.async_copy(g1_sp.at[sidx.at[j]], rows, gsem).wait()
        pltpu.sync_copy(rows, acc_sp.at[didx.at[j]], add=True)
    plsc.subcore_barrier()

    @pl.when(cid == 0)
    def _():
        pltpu.sync_copy(acc_sp.at[sl], p0_hbm.at[sl])

    @pl.when(cid == 1)
    def _():
        pltpu.sync_copy(acc_sp.at[sl], p1_hbm.at[sl])


def _agg1(h, src, dst):
    e = src.shape[0]
    e32 = e // 32
    ew = e // 16
    wv = 1000
    assert e32 % wv == 0 and wv % 8 == 0 and ew % 8 == 0
    return pl.kernel(
        functools.partial(_agg1_body, e32, ew, wv),
        out_type=[
            jax.ShapeDtypeStruct((NP, H1), jnp.float32),
            jax.ShapeDtypeStruct((NP, H1), jnp.float32),
            jax.ShapeDtypeStruct((NP,), jnp.float32),
        ],
        mesh=plsc.VectorSubcoreMesh(**_MESH),
        compiler_params=_SC_PARAMS_NOLAYOUT,
        scratch_types=[
            pltpu.VMEM_SHARED((NP, H1), jnp.float32),
            pltpu.VMEM_SHARED((NP, H1), jnp.float32),
            pltpu.VMEM_SHARED((NP,), jnp.float32),
            pltpu.VMEM((ZR, H1), jnp.float32),
            pltpu.VMEM((SLICE,), jnp.float32),
            pltpu.VMEM((ew,), jnp.float32),
            pltpu.VMEM((ew,), jnp.int32),
            pltpu.VMEM((SLICE, H1), jnp.float32),
            pltpu.VMEM((SLICE,), jnp.float32),
            pltpu.VMEM((5, wv), jnp.int32),
            pltpu.VMEM((5, wv), jnp.int32),
            pltpu.VMEM((wv, H1), jnp.float32),
            pltpu.SemaphoreType.DMA,
        ],
    )(h, src, dst)


def _tc2_body(pa_ref, pb_ref, h_ref, dis_ref, b1_ref, w2_ref, g2_ref):
    d = dis_ref[...]
    out1 = (pa_ref[...] + pb_ref[...] + h_ref[...] * d) * d + b1_ref[...]
    r = jnp.maximum(out1, 0.0)
    h2 = jnp.dot(r, w2_ref[...], preferred_element_type=jnp.float32)
    g2_ref[...] = h2 * d


def _tc2(pa, pb, h, dis, b1r, w2):
    blk = 2048
    return pl.pallas_call(
        _tc2_body,
        grid=(NP // blk,),
        in_specs=[
            pl.BlockSpec((blk, H1), lambda i: (i, 0)),
            pl.BlockSpec((blk, H1), lambda i: (i, 0)),
            pl.BlockSpec((blk, H1), lambda i: (i, 0)),
            pl.BlockSpec((blk, 1), lambda i: (i, 0)),
            pl.BlockSpec((1, H1), lambda i: (0, 0)),
            pl.BlockSpec((H1, 1), lambda i: (0, 0)),
        ],
        out_specs=pl.BlockSpec((blk, 1), lambda i: (i, 0)),
        out_shape=jax.ShapeDtypeStruct((NP, 1), jnp.float32),
    )(pa, pb, h, dis, b1r, w2)


def _agg2_body(ew, g2_hbm, src_hbm, dst_hbm, dis_hbm, b2_hbm, out_hbm,
               acc_sp, g2t, sidx, didx, vals, zbuf, gbuf, dbuf, b2buf):
    cid = lax.axis_index("c")
    sid = lax.axis_index("s")

    @pl.when(cid == 0)
    def _():
        sl = pl.ds(sid * SLICE, SLICE)
        _fill(zbuf, SLICE, 0.0)
        pltpu.sync_copy(zbuf, acc_sp.at[sl])
        pltpu.sync_copy(g2_hbm, g2t)
        pltpu.sync_copy(b2_hbm, b2buf)
        base = sid * ew
        pltpu.sync_copy(src_hbm.at[pl.ds(base, ew)], sidx)
        pltpu.sync_copy(dst_hbm.at[pl.ds(base, ew)], didx)

        def gat(j, _):
            s = pl.ds(j * L, L)
            vals[s] = plsc.load_gather(g2t, [sidx[s]])
            return 0
        lax.fori_loop(0, ew // L, gat, 0)
        plsc.subcore_barrier()
        pltpu.sync_copy(vals, acc_sp.at[didx], add=True)
        plsc.subcore_barrier()

        pltpu.sync_copy(acc_sp.at[sl], gbuf)
        pltpu.sync_copy(dis_hbm.at[sl], dbuf)
        b2v = b2buf[...]

        def fin(i, _):
            s = pl.ds(i * L, L)
            gg = g2t[pl.ds(sid * SLICE + i * L, L)]
            gbuf[s] = (gbuf[s] + gg) * dbuf[s] + b2v
            return 0
        lax.fori_loop(0, SLICE // L, fin, 0)
        pltpu.sync_copy(gbuf, out_hbm.at[sl])


def _agg2(g2, src, dst, dis, b2t):
    e = src.shape[0]
    ew = e // 16
    assert ew % 8 == 0 and ew % L == 0
    return pl.kernel(
        functools.partial(_agg2_body, ew),
        out_type=jax.ShapeDtypeStruct((NP,), jnp.float32),
        mesh=plsc.VectorSubcoreMesh(**_MESH),
        compiler_params=_SC_PARAMS_NOLAYOUT,
        scratch_types=[
            pltpu.VMEM_SHARED((NP,), jnp.float32),
            pltpu.VMEM((NP,), jnp.float32),
            pltpu.VMEM((ew,), jnp.int32),
            pltpu.VMEM((ew,), jnp.int32),
            pltpu.VMEM((ew,), jnp.float32),
            pltpu.VMEM((SLICE,), jnp.float32),
            pltpu.VMEM((SLICE,), jnp.float32),
            pltpu.VMEM((SLICE,), jnp.float32),
            pltpu.VMEM((L,), jnp.float32),
        ],
    )(g2, src, dst, dis, b2t)


def kernel(x, edge_index, W1, b1, W2, b2):
    src = edge_index[0]
    dst = edge_index[1]
    xp = jnp.pad(x, ((0, NP - N), (0, 0)))
    h = _tc1(xp, W1)
    p0, p1, dis = _agg1(h, src, dst)
    g2 = _tc2(p0, p1, h, dis.reshape(NP, 1),
              b1.reshape(1, H1), W2)
    b2t = jnp.tile(b2, L)
    outp = _agg2(g2.reshape(NP), src, dst, dis, b2t)
    return outp[:N].reshape(N, 1)


# speedup vs baseline: 1.0265x; 1.0265x over previous
import functools

import jax
import jax.numpy as jnp
from jax import lax
from jax.experimental import pallas as pl
from jax.experimental.pallas import tpu as pltpu
from jax.experimental.pallas import tpu_sc as plsc

N = 10000
NP = 10240
SLICE = NP // 16
H1 = 16
L = 16
ZR = 40

_MESH = dict(core_axis_name="c", subcore_axis_name="s")
_SC_PARAMS = pltpu.CompilerParams(use_tc_tiling_on_sc=False)
_SC_PARAMS_NOLAYOUT = pltpu.CompilerParams(use_tc_tiling_on_sc=False,
                                           needs_layout_passes=False)


def _fill(ref, n, value):
    def body(i, _):
        ref[pl.ds(i * L, L)] = jnp.full((L,), value, jnp.float32)
        return 0
    lax.fori_loop(0, n // L, body, 0)


def _tc1_body(x_ref, w_ref, h_ref):
    h_ref[...] = jnp.dot(x_ref[...], w_ref[...],
                         preferred_element_type=jnp.float32)


def _tc1(xp, w1):
    blk = 1024
    return pl.pallas_call(
        _tc1_body,
        grid=(NP // blk,),
        in_specs=[
            pl.BlockSpec((blk, xp.shape[1]), lambda i: (i, 0)),
            pl.BlockSpec((xp.shape[1], H1), lambda i: (0, 0)),
        ],
        out_specs=pl.BlockSpec((blk, H1), lambda i: (i, 0)),
        out_shape=jax.ShapeDtypeStruct((NP, H1), jnp.float32),
    )(xp, w1)


def _rsqrt16(x):
    i = plsc.bitcast(x, jnp.int32)
    y = plsc.bitcast(jnp.int32(0x5F3759DF) - (i >> 1), jnp.float32)
    for _ in range(3):
        y = y * (1.5 - 0.5 * x * y * y)
    return y


def _agg1_body(e32, ew, wv, h_hbm, src_hbm, dst_hbm, p0_hbm, p1_hbm, dis_hbm,
               g1_sp, acc_sp, deg_sp, zbuf, zvec, ones, idxb, hbuf, dvec,
               sidx, didx, rows, gsem):
    cid = lax.axis_index("c")
    sid = lax.axis_index("s")
    sl = pl.ds(sid * SLICE, SLICE)

    def z(i, _):
        zbuf[i] = jnp.zeros((L,), jnp.float32)
        return 0
    lax.fori_loop(0, ZR, z, 0)
    _fill(zvec, SLICE, 0.0)
    _fill(ones, ew, 1.0)
    for k in range(SLICE // ZR):
        pltpu.sync_copy(zbuf, acc_sp.at[pl.ds(sid * SLICE + k * ZR, ZR)])
    pltpu.sync_copy(zvec, deg_sp.at[sl])

    pltpu.sync_copy(dst_hbm.at[pl.ds(sid * ew, ew)], idxb)
    plsc.subcore_barrier()
    pltpu.sync_copy(ones, deg_sp.at[idxb], add=True)
    plsc.subcore_barrier()

    pltpu.sync_copy(deg_sp.at[sl], dvec)
    pltpu.sync_copy(h_hbm.at[sl], hbuf)

    def mkdis(i, _):
        s = pl.ds(i * L, L)
        dvec[s] = _rsqrt16(dvec[s] + 1.0)
        return 0
    lax.fori_loop(0, SLICE // L, mkdis, 0)

    @pl.when(cid == 0)
    def _():
        pltpu.sync_copy(dvec, dis_hbm.at[sl])

    gdn = lax.GatherDimensionNumbers(offset_dims=(), collapsed_slice_dims=(0,),
                                     start_index_map=(0,))

    def scale(i, _):
        d16 = dvec[pl.ds(i * L, L)]
        for r in range(L):
            dsplat = lax.gather(
                d16, jnp.full((L, 1), r, jnp.int32), dimension_numbers=gdn,
                slice_sizes=(1,),
                mode=lax.GatherScatterMode.PROMISE_IN_BOUNDS)
            hbuf[i * L + r] = hbuf[i * L + r] * dsplat
        return 0
    lax.fori_loop(0, SLICE // L, scale, 0)
    pltpu.sync_copy(hbuf, g1_sp.at[sl])
    plsc.subcore_barrier()

    base = (cid * 16 + sid) * e32
    for j in range(e32 // wv):
        pltpu.sync_copy(src_hbm.at[pl.ds(base + j * wv, wv)], sidx.at[j])
        pltpu.sync_copy(dst_hbm.at[pl.ds(base + j * wv, wv)], didx.at[j])
    for j in range(e32 // wv):
        pltpu.async_copy(g1_sp.at[sidx.at[j]], rows, gsem).wait()
        pltpu.sync_copy(rows, acc_sp.at[didx.at[j]], add=True)
    plsc.subcore_barrier()

    @pl.when(cid == 0)
    def _():
        pltpu.sync_copy(acc_sp.at[sl], p0_hbm.at[sl])

    @pl.when(cid == 1)
    def _():
        pltpu.sync_copy(acc_sp.at[sl], p1_hbm.at[sl])


def _agg1(h, src, dst):
    e = src.shape[0]
    e32 = e // 32
    ew = e // 16
    wv = 1000
    assert e32 % wv == 0 and wv % 8 == 0 and ew % 8 == 0
    return pl.kernel(
        functools.partial(_agg1_body, e32, ew, wv),
        out_type=[
            jax.ShapeDtypeStruct((NP, H1), jnp.float32),
            jax.ShapeDtypeStruct((NP, H1), jnp.float32),
            jax.ShapeDtypeStruct((NP,), jnp.float32),
        ],
        mesh=plsc.VectorSubcoreMesh(**_MESH),
        compiler_params=_SC_PARAMS_NOLAYOUT,
        scratch_types=[
            pltpu.VMEM_SHARED((NP, H1), jnp.float32),
            pltpu.VMEM_SHARED((NP, H1), jnp.float32),
            pltpu.VMEM_SHARED((NP,), jnp.float32),
            pltpu.VMEM((ZR, H1), jnp.float32),
            pltpu.VMEM((SLICE,), jnp.float32),
            pltpu.VMEM((ew,), jnp.float32),
            pltpu.VMEM((ew,), jnp.int32),
            pltpu.VMEM((SLICE, H1), jnp.float32),
            pltpu.VMEM((SLICE,), jnp.float32),
            pltpu.VMEM((5, wv), jnp.int32),
            pltpu.VMEM((5, wv), jnp.int32),
            pltpu.VMEM((wv, H1), jnp.float32),
            pltpu.SemaphoreType.DMA,
        ],
    )(h, src, dst)


def _tc2_body(pa_ref, pb_ref, h_ref, dis_ref, b1_ref, w2_ref, g2_ref):
    d = dis_ref[...]
    out1 = (pa_ref[...] + pb_ref[...] + h_ref[...] * d) * d + b1_ref[...]
    r = jnp.maximum(out1, 0.0)
    h2 = jnp.dot(r, w2_ref[...], preferred_element_type=jnp.float32)
    g2_ref[...] = h2 * d


def _tc2(pa, pb, h, dis, b1r, w2):
    blk = 2048
    return pl.pallas_call(
        _tc2_body,
        grid=(NP // blk,),
        in_specs=[
            pl.BlockSpec((blk, H1), lambda i: (i, 0)),
            pl.BlockSpec((blk, H1), lambda i: (i, 0)),
            pl.BlockSpec((blk, H1), lambda i: (i, 0)),
            pl.BlockSpec((blk, 1), lambda i: (i, 0)),
            pl.BlockSpec((1, H1), lambda i: (0, 0)),
            pl.BlockSpec((H1, 1), lambda i: (0, 0)),
        ],
        out_specs=pl.BlockSpec((blk, 1), lambda i: (i, 0)),
        out_shape=jax.ShapeDtypeStruct((NP, 1), jnp.float32),
    )(pa, pb, h, dis, b1r, w2)


def _agg2_body(ew, g2_hbm, src_hbm, dst_hbm, dis_hbm, b2_hbm, out_hbm,
               acc_sp, g2t, sidx, didx, vals, zbuf, gbuf, dbuf, b2buf):
    cid = lax.axis_index("c")
    sid = lax.axis_index("s")

    @pl.when(cid == 0)
    def _():
        sl = pl.ds(sid * SLICE, SLICE)
        _fill(zbuf, SLICE, 0.0)
        pltpu.sync_copy(zbuf, acc_sp.at[sl])
        pltpu.sync_copy(g2_hbm, g2t)
        pltpu.sync_copy(b2_hbm, b2buf)
        base = sid * ew
        pltpu.sync_copy(src_hbm.at[pl.ds(base, ew)], sidx)
        pltpu.sync_copy(dst_hbm.at[pl.ds(base, ew)], didx)

        def gat(j, _):
            s = pl.ds(j * L, L)
            vals[s] = plsc.load_gather(g2t, [sidx[s]])
            return 0
        lax.fori_loop(0, ew // L, gat, 0)
        plsc.subcore_barrier()
        pltpu.sync_copy(vals, acc_sp.at[didx], add=True)
        plsc.subcore_barrier()

        pltpu.sync_copy(acc_sp.at[sl], gbuf)
        pltpu.sync_copy(dis_hbm.at[sl], dbuf)
        b2v = b2buf[...]

        def fin(i, _):
            s = pl.ds(i * L, L)
            gg = g2t[pl.ds(sid * SLICE + i * L, L)]
            gbuf[s] = (gbuf[s] + gg) * dbuf[s] + b2v
            return 0
        lax.fori_loop(0, SLICE // L, fin, 0)
        pltpu.sync_copy(gbuf, out_hbm.at[sl])


def _agg2(g2, src, dst, dis, b2t):
    e = src.shape[0]
    ew = e // 16
    assert ew % 8 == 0 and ew % L == 0
    return pl.kernel(
        functools.partial(_agg2_body, ew),
        out_type=jax.ShapeDtypeStruct((NP,), jnp.float32),
        mesh=plsc.VectorSubcoreMesh(**_MESH),
        compiler_params=_SC_PARAMS_NOLAYOUT,
        scratch_types=[
            pltpu.VMEM_SHARED((NP,), jnp.float32),
            pltpu.VMEM((NP,), jnp.float32),
            pltpu.VMEM((ew,), jnp.int32),
            pltpu.VMEM((ew,), jnp.int32),
            pltpu.VMEM((ew,), jnp.float32),
            pltpu.VMEM((SLICE,), jnp.float32),
            pltpu.VMEM((SLICE,), jnp.float32),
            pltpu.VMEM((SLICE,), jnp.float32),
            pltpu.VMEM((L,), jnp.float32),
        ],
    )(g2, src, dst, dis, b2t)


def kernel(x, edge_index, W1, b1, W2, b2):
    src = edge_index[0]
    dst = edge_index[1]
    xp = jnp.pad(x, ((0, NP - N), (0, 0)))
    h = _tc1(xp, W1)
    p0, p1, dis = _agg1(h, src, dst)
    g2 = _tc2(p0, p1, h, dis.reshape(NP, 1),
              b1.reshape(1, H1), W2)
    b2t = jnp.tile(b2, L)
    outp = _agg2(g2.reshape(NP), src, dst, dis, b2t)
    return outp[:N].reshape(N, 1)


# speedup vs baseline: 1.0921x; 1.0640x over previous
import functools

import jax
import jax.numpy as jnp
from jax import lax
from jax.experimental import pallas as pl
from jax.experimental.pallas import tpu as pltpu
from jax.experimental.pallas import tpu_sc as plsc

N = 10000
NP = 10240
SLICE = NP // 16
H1 = 16
L = 16

_MESH = dict(core_axis_name="c", subcore_axis_name="s")
_SC_PARAMS = pltpu.CompilerParams(use_tc_tiling_on_sc=False)
_SC_PARAMS_NOLAYOUT = pltpu.CompilerParams(use_tc_tiling_on_sc=False,
                                           needs_layout_passes=False)


def _fill(ref, n, value):
    def body(i, _):
        ref[pl.ds(i * L, L)] = jnp.full((L,), value, jnp.float32)
        return 0
    lax.fori_loop(0, n // L, body, 0)


def _deg_body(ew, dst_hbm, deg_hbm, deg_sp, zbuf, ones, idxb):
    cid = lax.axis_index("c")
    sid = lax.axis_index("s")

    @pl.when(cid == 0)
    def _():
        _fill(zbuf, SLICE, 0.0)
        _fill(ones, ew, 1.0)
        sl = pl.ds(sid * SLICE, SLICE)
        pltpu.sync_copy(zbuf, deg_sp.at[sl])
        pltpu.sync_copy(dst_hbm.at[pl.ds(sid * ew, ew)], idxb)
        plsc.subcore_barrier()
        pltpu.sync_copy(ones, deg_sp.at[idxb], add=True)
        plsc.subcore_barrier()
        pltpu.sync_copy(deg_sp.at[sl], deg_hbm.at[sl])


def _deg_hist(dst):
    e = dst.shape[0]
    ew = e // 16
    assert ew % 8 == 0
    return pl.kernel(
        functools.partial(_deg_body, ew),
        out_type=jax.ShapeDtypeStruct((NP,), jnp.float32),
        mesh=plsc.VectorSubcoreMesh(**_MESH),
        compiler_params=_SC_PARAMS,
        scratch_types=[
            pltpu.VMEM_SHARED((NP,), jnp.float32),
            pltpu.VMEM((SLICE,), jnp.float32),
            pltpu.VMEM((ew,), jnp.float32),
            pltpu.VMEM((ew,), jnp.int32),
        ],
    )(dst)


def _tc1_body(x_ref, w_ref, deg_ref, g1_ref, dis_ref):
    h = jnp.dot(x_ref[...], w_ref[...], preferred_element_type=jnp.float32)
    d = lax.rsqrt(deg_ref[...] + 1.0)
    dis_ref[...] = d
    g1_ref[...] = h * d


def _tc1(xp, w1, degc):
    blk = 1024
    return pl.pallas_call(
        _tc1_body,
        grid=(NP // blk,),
        in_specs=[
            pl.BlockSpec((blk, xp.shape[1]), lambda i: (i, 0)),
            pl.BlockSpec((xp.shape[1], H1), lambda i: (0, 0)),
            pl.BlockSpec((blk, 1), lambda i: (i, 0)),
        ],
        out_specs=[
            pl.BlockSpec((blk, H1), lambda i: (i, 0)),
            pl.BlockSpec((blk, 1), lambda i: (i, 0)),
        ],
        out_shape=[
            jax.ShapeDtypeStruct((NP, H1), jnp.float32),
            jax.ShapeDtypeStruct((NP, 1), jnp.float32),
        ],
    )(xp, w1, degc)


def _agg1_body(e32, g1_hbm, src_hbm, dst_hbm, p0_hbm, p1_hbm,
               acc_sp, zbuf, sidx, didx, rows, gsem):
    cid = lax.axis_index("c")
    sid = lax.axis_index("s")
    sl = pl.ds(sid * SLICE, SLICE)

    def z(i, _):
        zbuf[i] = jnp.zeros((L,), jnp.float32)
        return 0
    lax.fori_loop(0, SLICE, z, 0)
    pltpu.sync_copy(zbuf, acc_sp.at[sl])

    base = (cid * 16 + sid) * e32
    pltpu.sync_copy(src_hbm.at[pl.ds(base, e32)], sidx)
    pltpu.sync_copy(dst_hbm.at[pl.ds(base, e32)], didx)
    plsc.subcore_barrier()
    pltpu.async_copy(g1_hbm.at[sidx], rows, gsem).wait()
    pltpu.sync_copy(rows, acc_sp.at[didx], add=True)
    plsc.subcore_barrier()

    @pl.when(cid == 0)
    def _():
        pltpu.sync_copy(acc_sp.at[sl], p0_hbm.at[sl])

    @pl.when(cid == 1)
    def _():
        pltpu.sync_copy(acc_sp.at[sl], p1_hbm.at[sl])


def _agg1(g1, src, dst):
    e = src.shape[0]
    e32 = e // 32
    assert e32 % 8 == 0
    return pl.kernel(
        functools.partial(_agg1_body, e32),
        out_type=[
            jax.ShapeDtypeStruct((NP, H1), jnp.float32),
            jax.ShapeDtypeStruct((NP, H1), jnp.float32),
        ],
        mesh=plsc.VectorSubcoreMesh(**_MESH),
        compiler_params=_SC_PARAMS,
        scratch_types=[
            pltpu.VMEM_SHARED((NP, H1), jnp.float32),
            pltpu.VMEM((SLICE, H1), jnp.float32),
            pltpu.VMEM((e32,), jnp.int32),
            pltpu.VMEM((e32,), jnp.int32),
            pltpu.VMEM((e32, H1), jnp.float32),
            pltpu.SemaphoreType.DMA,
        ],
    )(g1, src, dst)


def _tc2_body(pa_ref, pb_ref, g1_ref, dis_ref, b1_ref, w2_ref, g2_ref):
    d = dis_ref[...]
    out1 = (pa_ref[...] + pb_ref[...] + g1_ref[...]) * d + b1_ref[...]
    r = jnp.maximum(out1, 0.0)
    h2 = jnp.dot(r, w2_ref[...], preferred_element_type=jnp.float32)
    g2_ref[...] = h2 * d


def _tc2(pa, pb, g1, dis, b1r, w2):
    blk = 2048
    return pl.pallas_call(
        _tc2_body,
        grid=(NP // blk,),
        in_specs=[
            pl.BlockSpec((blk, H1), lambda i: (i, 0)),
            pl.BlockSpec((blk, H1), lambda i: (i, 0)),
            pl.BlockSpec((blk, H1), lambda i: (i, 0)),
            pl.BlockSpec((blk, 1), lambda i: (i, 0)),
            pl.BlockSpec((1, H1), lambda i: (0, 0)),
            pl.BlockSpec((H1, 1), lambda i: (0, 0)),
        ],
        out_specs=pl.BlockSpec((blk, 1), lambda i: (i, 0)),
        out_shape=jax.ShapeDtypeStruct((NP, 1), jnp.float32),
    )(pa, pb, g1, dis, b1r, w2)


def _agg2_body(ew, g2_hbm, src_hbm, dst_hbm, dis_hbm, b2_hbm, out_hbm,
               acc_sp, g2t, sidx, didx, vals, zbuf, gbuf, dbuf, b2buf):
    cid = lax.axis_index("c")
    sid = lax.axis_index("s")

    @pl.when(cid == 0)
    def _():
        sl = pl.ds(sid * SLICE, SLICE)
        _fill(zbuf, SLICE, 0.0)
        pltpu.sync_copy(zbuf, acc_sp.at[sl])
        pltpu.sync_copy(g2_hbm, g2t)
        pltpu.sync_copy(b2_hbm, b2buf)
        base = sid * ew
        pltpu.sync_copy(src_hbm.at[pl.ds(base, ew)], sidx)
        pltpu.sync_copy(dst_hbm.at[pl.ds(base, ew)], didx)

        def gat(j, _):
            s = pl.ds(j * L, L)
            vals[s] = plsc.load_gather(g2t, [sidx[s]])
            return 0
        lax.fori_loop(0, ew // L, gat, 0)
        plsc.subcore_barrier()
        pltpu.sync_copy(vals, acc_sp.at[didx], add=True)
        plsc.subcore_barrier()

        pltpu.sync_copy(acc_sp.at[sl], gbuf)
        pltpu.sync_copy(dis_hbm.at[sl], dbuf)
        b2v = b2buf[...]

        def fin(i, _):
            s = pl.ds(i * L, L)
            gg = g2t[pl.ds(sid * SLICE + i * L, L)]
            gbuf[s] = (gbuf[s] + gg) * dbuf[s] + b2v
            return 0
        lax.fori_loop(0, SLICE // L, fin, 0)
        pltpu.sync_copy(gbuf, out_hbm.at[sl])


def _agg2(g2, src, dst, dis, b2t):
    e = src.shape[0]
    ew = e // 16
    assert ew % 8 == 0 and ew % L == 0
    return pl.kernel(
        functools.partial(_agg2_body, ew),
        out_type=jax.ShapeDtypeStruct((NP,), jnp.float32),
        mesh=plsc.VectorSubcoreMesh(**_MESH),
        compiler_params=_SC_PARAMS_NOLAYOUT,
        scratch_types=[
            pltpu.VMEM_SHARED((NP,), jnp.float32),
            pltpu.VMEM((NP,), jnp.float32),
            pltpu.VMEM((ew,), jnp.int32),
            pltpu.VMEM((ew,), jnp.int32),
            pltpu.VMEM((ew,), jnp.float32),
            pltpu.VMEM((SLICE,), jnp.float32),
            pltpu.VMEM((SLICE,), jnp.float32),
            pltpu.VMEM((SLICE,), jnp.float32),
            pltpu.VMEM((L,), jnp.float32),
        ],
    )(g2, src, dst, dis, b2t)


def kernel(x, edge_index, W1, b1, W2, b2):
    src = edge_index[0]
    dst = edge_index[1]
    xp = jnp.pad(x, ((0, NP - N), (0, 0)))
    deg = _deg_hist(dst)
    g1, dis = _tc1(xp, W1, deg.reshape(NP, 1))
    p0, p1 = _agg1(g1, src, dst)
    g2 = _tc2(p0, p1, g1, dis, b1.reshape(1, H1), W2)
    b2t = jnp.tile(b2, L)
    outp = _agg2(g2.reshape(NP), src, dst, dis.reshape(NP), b2t)
    return outp[:N].reshape(N, 1)
